# baseline (speedup 1.0000x reference)
import jax
import jax.numpy as jnp
from jax import lax
from jax.experimental import pallas as pl
from jax.experimental.pallas import tpu as pltpu
from jax.experimental.pallas import tpu_sc as plsc

B = 32
N = 500000
GR = 1024
GC = 1024
CELLS = GR * GC
NC = 2
NS = 16
BPC = B // NC
CH = 2000
NCH = N // CH
NV = CH // 16
DUMP = 64
SLICE = CELLS // NS
DB = 16384


def _body(in_hbm, ks_hbm, perm_hbm, out_hbm,
          kbuf, pbuf, nbuf, cellbuf, valbuf, sbuf, dbuf, spin, splane, sem):
    c = lax.axis_index("c")
    s = lax.axis_index("s")

    def batch_body(k, _):
        b = c * BPC + k
        boff = b * N

        def zfill(t, _):
            dbuf[pl.ds(t * 16, 16)] = jnp.zeros((16,), jnp.float32)
            return 0
        lax.fori_loop(0, DB // 16, zfill, 0)

        def zdma(t, _):
            pltpu.sync_copy(dbuf, splane.at[pl.ds(s * SLICE + t * DB, DB)])
            return 0
        lax.fori_loop(0, SLICE // DB, zdma, 0)

        def stage(t, _):
            idx = s + NS * t

            @pl.when(idx < NCH)
            def _():
                off = idx * CH
                pltpu.sync_copy(in_hbm.at[pl.ds(boff + off, CH)], sbuf)
                pltpu.sync_copy(sbuf, spin.at[pl.ds(off, CH)])
            return 0
        lax.fori_loop(0, (NCH + NS - 1) // NS, stage, 0)
        plsc.subcore_barrier()

        def chunk(t, _):
            ci = s + NS * t

            @pl.when(ci < NCH)
            def _():
                off = boff + ci * CH
                pltpu.sync_copy(ks_hbm.at[pl.ds(off, CH + 16)], kbuf)
                pltpu.sync_copy(perm_hbm.at[pl.ds(off, CH)], pbuf)

                def vec(v, _):
                    cur = kbuf[pl.ds(v * 16, 16)]
                    nxt = plsc.load_gather(
                        kbuf, [v * 16 + 1 + lax.iota(jnp.int32, 16)])
                    rend = cur != nxt
                    cell = jnp.bitwise_and(cur, CELLS - 1)
                    dump = CELLS + jnp.bitwise_and(
                        lax.iota(jnp.int32, 16), DUMP - 1)
                    cellbuf[pl.ds(v * 16, 16)] = jnp.where(rend, cell, dump)
                    nbuf[pl.ds(v * 16, 16)] = pbuf[pl.ds(v * 16, 16)] - boff
                    return 0
                lax.fori_loop(0, NV, vec, 0)
                pltpu.async_copy(spin.at[nbuf], valbuf, sem).wait()
                pltpu.sync_copy(valbuf, splane.at[cellbuf])
            return 0
        lax.fori_loop(0, (NCH + NS - 1) // NS, chunk, 0)
        plsc.subcore_barrier()

        def drain(t, _):
            pltpu.sync_copy(splane.at[pl.ds(s * SLICE + t * DB, DB)], dbuf)
            pltpu.sync_copy(dbuf, out_hbm.at[pl.ds(b * CELLS + s * SLICE
                                                   + t * DB, DB)])
            return 0
        lax.fori_loop(0, SLICE // DB, drain, 0)
        return 0
    lax.fori_loop(0, BPC, batch_body, 0)


_regrid = pl.kernel(
    _body,
    out_type=jax.ShapeDtypeStruct((B * CELLS,), jnp.float32),
    mesh=plsc.VectorSubcoreMesh(
        core_axis_name="c", subcore_axis_name="s",
        num_cores=NC, num_subcores=NS),
    compiler_params=pltpu.CompilerParams(needs_layout_passes=False),
    scratch_types=[
        pltpu.VMEM((CH + 16,), jnp.int32),
        pltpu.VMEM((CH,), jnp.int32),
        pltpu.VMEM((CH,), jnp.int32),
        pltpu.VMEM((CH,), jnp.int32),
        pltpu.VMEM((CH,), jnp.float32),
        pltpu.VMEM((CH,), jnp.float32),
        pltpu.VMEM((DB,), jnp.float32),
        pltpu.VMEM_SHARED((N,), jnp.float32),
        pltpu.VMEM_SHARED((CELLS + DUMP,), jnp.float32),
        pltpu.SemaphoreType.DMA,
    ],
)


def kernel(inputs, row_indices, col_indices):
    bidx = jnp.repeat(jnp.arange(B, dtype=jnp.int32), N)
    key = ((bidx << 20) | (jnp.tile(row_indices, B) << 10)
           | jnp.tile(col_indices, B))
    iota = lax.iota(jnp.int32, B * N)
    ks, pv = lax.sort_key_val(
        key, lax.bitcast_convert_type(iota, jnp.float32), is_stable=False)
    perm = lax.bitcast_convert_type(pv, jnp.int32)
    ksp = jnp.concatenate([ks, jnp.full((16,), -1, jnp.int32)])
    out = _regrid(inputs.reshape(-1), ksp, perm)
    return out.reshape(B, GR, GC, 1)


# speedup vs baseline: 1.0343x; 1.0343x over previous
import jax
import jax.numpy as jnp
from jax import lax
from jax.experimental import pallas as pl
from jax.experimental.pallas import tpu as pltpu
from jax.experimental.pallas import tpu_sc as plsc

B = 32
N = 500000
GR = 1024
GC = 1024
CELLS = GR * GC
NC = 2
NS = 16
BPC = B // NC
CH = 2000
NCH = N // CH
NV = CH // 16
DUMP = 64
SLICE = CELLS // NS
DB = 32768


def _body(ks_hbm, vs_hbm, out_hbm,
          kbuf, cellbuf, valbuf, dbuf, splane, sem):
    c = lax.axis_index("c")
    s = lax.axis_index("s")

    def zfill(t, _):
        dbuf[pl.ds(t * 16, 16)] = jnp.zeros((16,), jnp.float32)
        return 0
    lax.fori_loop(0, DB // 16, zfill, 0)

    def batch_body(k, _):
        b = c * BPC + k
        boff = b * N

        def zdma(t, _):
            pltpu.sync_copy(dbuf, splane.at[pl.ds(s * SLICE + t * DB, DB)])
            return 0
        lax.fori_loop(0, SLICE // DB, zdma, 0)
        plsc.subcore_barrier()

        def chunk(t, _):
            ci = s + NS * t

            @pl.when(ci < NCH)
            def _():
                off = boff + ci * CH
                pltpu.sync_copy(ks_hbm.at[pl.ds(off, CH + 16)], kbuf)
                pltpu.sync_copy(vs_hbm.at[pl.ds(off, CH)], valbuf)

                def vec(v, _):
                    cur = kbuf[pl.ds(v * 16, 16)]
                    nxt = plsc.load_gather(
                        kbuf, [v * 16 + 1 + lax.iota(jnp.int32, 16)])
                    rend = cur != nxt
                    cell = jnp.bitwise_and(cur, CELLS - 1)
                    dump = CELLS + jnp.bitwise_and(
                        lax.iota(jnp.int32, 16), DUMP - 1)
                    cellbuf[pl.ds(v * 16, 16)] = jnp.where(rend, cell, dump)
                    return 0
                lax.fori_loop(0, NV, vec, 0)
                pltpu.sync_copy(valbuf, splane.at[cellbuf])
            return 0
        lax.fori_loop(0, (NCH + NS - 1) // NS, chunk, 0)
        plsc.subcore_barrier()

        def drain(t, _):
            pltpu.sync_copy(splane.at[pl.ds(s * SLICE + t * DB, DB)], dbuf)
            pltpu.sync_copy(dbuf, out_hbm.at[pl.ds(b * CELLS + s * SLICE
                                                   + t * DB, DB)])
            return 0
        lax.fori_loop(0, SLICE // DB, drain, 0)

        def zfill2(t, _):
            dbuf[pl.ds(t * 16, 16)] = jnp.zeros((16,), jnp.float32)
            return 0
        lax.fori_loop(0, DB // 16, zfill2, 0)
        return 0
    lax.fori_loop(0, BPC, batch_body, 0)


_regrid = pl.kernel(
    _body,
    out_type=jax.ShapeDtypeStruct((B * CELLS,), jnp.float32),
    mesh=plsc.VectorSubcoreMesh(
        core_axis_name="c", subcore_axis_name="s",
        num_cores=NC, num_subcores=NS),
    compiler_params=pltpu.CompilerParams(needs_layout_passes=False),
    scratch_types=[
        pltpu.VMEM((CH + 16,), jnp.int32),
        pltpu.VMEM((CH,), jnp.int32),
        pltpu.VMEM((CH,), jnp.float32),
        pltpu.VMEM((DB,), jnp.float32),
        pltpu.VMEM_SHARED((CELLS + DUMP,), jnp.float32),
        pltpu.SemaphoreType.DMA,
    ],
)


def kernel(inputs, row_indices, col_indices):
    bidx = jnp.repeat(jnp.arange(B, dtype=jnp.int32), N)
    key = ((bidx << 20) | (jnp.tile(row_indices, B) << 10)
           | jnp.tile(col_indices, B))
    ks, vs = lax.sort_key_val(key, inputs.reshape(-1), is_stable=False)
    ksp = jnp.concatenate([ks, jnp.full((16,), -1, jnp.int32)])
    out = _regrid(ksp, vs)
    return out.reshape(B, GR, GC, 1)


# speedup vs baseline: 1.0377x; 1.0033x over previous
import jax
import jax.numpy as jnp
from jax import lax
from jax.experimental import pallas as pl
from jax.experimental.pallas import tpu as pltpu
from jax.experimental.pallas import tpu_sc as plsc

B = 32
N = 500000
GR = 1024
GC = 1024
CELLS = GR * GC
NC = 2
NS = 16
BPC = B // NC
CH = 2000
NCH = N // CH
NV = CH // 16
DUMP = 64
SLICE = CELLS // NS
DB = 32768


def _body(ks_hbm, vs_hbm, out_hbm,
          kbuf, cellbuf, valbuf, dbuf, splane, sem):
    c = lax.axis_index("c")
    s = lax.axis_index("s")

    def zfill(t, _):
        dbuf[pl.ds(t * 16, 16)] = jnp.zeros((16,), jnp.float32)
        return 0
    lax.fori_loop(0, DB // 16, zfill, 0)

    def batch_body(k, _):
        b = c * BPC + k
        boff = b * N

        def zdma(t, _):
            pltpu.sync_copy(dbuf, splane.at[pl.ds(s * SLICE + t * DB, DB)])
            return 0
        lax.fori_loop(0, SLICE // DB, zdma, 0)
        plsc.subcore_barrier()

        def chunk(t, _):
            ci = s + NS * t

            @pl.when(ci < NCH)
            def _():
                off = boff + ci * CH
                last = jnp.logical_and(b == B - 1, ci == NCH - 1)

                @pl.when(jnp.logical_not(last))
                def _():
                    pltpu.sync_copy(ks_hbm.at[pl.ds(off, CH + 16)], kbuf)

                @pl.when(last)
                def _():
                    pltpu.sync_copy(ks_hbm.at[pl.ds(off, CH)],
                                    kbuf.at[pl.ds(0, CH)])
                    kbuf[pl.ds(CH, 16)] = jnp.full((16,), -1, jnp.int32)
                pltpu.sync_copy(vs_hbm.at[pl.ds(off, CH)], valbuf)

                def vec(v, _):
                    cur = kbuf[pl.ds(v * 16, 16)]
                    nxt = plsc.load_gather(
                        kbuf, [v * 16 + 1 + lax.iota(jnp.int32, 16)])
                    rend = cur != nxt
                    cell = jnp.bitwise_and(cur, CELLS - 1)
                    dump = CELLS + jnp.bitwise_and(
                        lax.iota(jnp.int32, 16), DUMP - 1)
                    cellbuf[pl.ds(v * 16, 16)] = jnp.where(rend, cell, dump)
                    return 0
                lax.fori_loop(0, NV, vec, 0)
                pltpu.sync_copy(valbuf, splane.at[cellbuf])
            return 0
        lax.fori_loop(0, (NCH + NS - 1) // NS, chunk, 0)
        plsc.subcore_barrier()

        def drain(t, _):
            pltpu.sync_copy(splane.at[pl.ds(s * SLICE + t * DB, DB)], dbuf)
            pltpu.sync_copy(dbuf, out_hbm.at[pl.ds(b * CELLS + s * SLICE
                                                   + t * DB, DB)])
            return 0
        lax.fori_loop(0, SLICE // DB, drain, 0)

        def zfill2(t, _):
            dbuf[pl.ds(t * 16, 16)] = jnp.zeros((16,), jnp.float32)
            return 0
        lax.fori_loop(0, DB // 16, zfill2, 0)
        return 0
    lax.fori_loop(0, BPC, batch_body, 0)


_regrid = pl.kernel(
    _body,
    out_type=jax.ShapeDtypeStruct((B * CELLS,), jnp.float32),
    mesh=plsc.VectorSubcoreMesh(
        core_axis_name="c", subcore_axis_name="s",
        num_cores=NC, num_subcores=NS),
    compiler_params=pltpu.CompilerParams(needs_layout_passes=False),
    scratch_types=[
        pltpu.VMEM((CH + 16,), jnp.int32),
        pltpu.VMEM((CH,), jnp.int32),
        pltpu.VMEM((CH,), jnp.float32),
        pltpu.VMEM((DB,), jnp.float32),
        pltpu.VMEM_SHARED((CELLS + DUMP,), jnp.float32),
        pltpu.SemaphoreType.DMA,
    ],
)


def kernel(inputs, row_indices, col_indices):
    bidx = jnp.repeat(jnp.arange(B, dtype=jnp.int32), N)
    key = ((bidx << 20) | (jnp.tile(row_indices, B) << 10)
           | jnp.tile(col_indices, B))
    ks, vs = lax.sort_key_val(key, inputs.reshape(-1), is_stable=False)
    out = _regrid(ks, vs)
    return out.reshape(B, GR, GC, 1)


# speedup vs baseline: 1.0413x; 1.0034x over previous
import jax
import jax.numpy as jnp
from jax import lax
from jax.experimental import pallas as pl
from jax.experimental.pallas import tpu as pltpu
from jax.experimental.pallas import tpu_sc as plsc

B = 32
N = 500000
GR = 1024
GC = 1024
CELLS = GR * GC
NC = 2
NS = 16
BPC = B // NC
CH = 4000
NCH = N // CH
NV = CH // 16
DUMP = 64
SLICE = CELLS // NS
DB = 32768


def _body(ks_hbm, vs_hbm, out_hbm,
          kbuf, cellbuf, valbuf, dbuf, splane, sem):
    c = lax.axis_index("c")
    s = lax.axis_index("s")

    def zfill(t, _):
        dbuf[pl.ds(t * 16, 16)] = jnp.zeros((16,), jnp.float32)
        return 0
    lax.fori_loop(0, DB // 16, zfill, 0)

    def batch_body(k, _):
        b = c * BPC + k
        boff = b * N

        def zdma(t, _):
            pltpu.sync_copy(dbuf, splane.at[pl.ds(s * SLICE + t * DB, DB)])
            return 0
        lax.fori_loop(0, SLICE // DB, zdma, 0)
        plsc.subcore_barrier()

        def chunk(t, _):
            ci = s + NS * t

            @pl.when(ci < NCH)
            def _():
                off = boff + ci * CH
                last = jnp.logical_and(b == B - 1, ci == NCH - 1)

                @pl.when(jnp.logical_not(last))
                def _():
                    pltpu.sync_copy(ks_hbm.at[pl.ds(off, CH + 16)], kbuf)

                @pl.when(last)
                def _():
                    pltpu.sync_copy(ks_hbm.at[pl.ds(off, CH)],
                                    kbuf.at[pl.ds(0, CH)])
                    kbuf[pl.ds(CH, 16)] = jnp.full((16,), -1, jnp.int32)
                pltpu.sync_copy(vs_hbm.at[pl.ds(off, CH)], valbuf)

                def vec(v, _):
                    cur = kbuf[pl.ds(v * 16, 16)]
                    nxt = plsc.load_gather(
                        kbuf, [v * 16 + 1 + lax.iota(jnp.int32, 16)])
                    rend = cur != nxt
                    cell = jnp.bitwise_and(cur, CELLS - 1)
                    dump = CELLS + jnp.bitwise_and(
                        lax.iota(jnp.int32, 16), DUMP - 1)
                    cellbuf[pl.ds(v * 16, 16)] = jnp.where(rend, cell, dump)
                    return 0
                lax.fori_loop(0, NV, vec, 0)
                pltpu.sync_copy(valbuf, splane.at[cellbuf])
            return 0
        lax.fori_loop(0, (NCH + NS - 1) // NS, chunk, 0)
        plsc.subcore_barrier()

        def drain(t, _):
            pltpu.sync_copy(splane.at[pl.ds(s * SLICE + t * DB, DB)], dbuf)
            pltpu.sync_copy(dbuf, out_hbm.at[pl.ds(b * CELLS + s * SLICE
                                                   + t * DB, DB)])
            return 0
        lax.fori_loop(0, SLICE // DB, drain, 0)

        def zfill2(t, _):
            dbuf[pl.ds(t * 16, 16)] = jnp.zeros((16,), jnp.float32)
            return 0
        lax.fori_loop(0, DB // 16, zfill2, 0)
        return 0
    lax.fori_loop(0, BPC, batch_body, 0)


_regrid = pl.kernel(
    _body,
    out_type=jax.ShapeDtypeStruct((B * CELLS,), jnp.float32),
    mesh=plsc.VectorSubcoreMesh(
        core_axis_name="c", subcore_axis_name="s",
        num_cores=NC, num_subcores=NS),
    compiler_params=pltpu.CompilerParams(needs_layout_passes=False),
    scratch_types=[
        pltpu.VMEM((CH + 16,), jnp.int32),
        pltpu.VMEM((CH,), jnp.int32),
        pltpu.VMEM((CH,), jnp.float32),
        pltpu.VMEM((DB,), jnp.float32),
        pltpu.VMEM_SHARED((CELLS + DUMP,), jnp.float32),
        pltpu.SemaphoreType.DMA,
    ],
)


def kernel(inputs, row_indices, col_indices):
    bidx = jnp.repeat(jnp.arange(B, dtype=jnp.int32), N)
    key = ((bidx << 20) | (jnp.tile(row_indices, B) << 10)
           | jnp.tile(col_indices, B))
    ks, vs = lax.sort_key_val(key, inputs.reshape(-1), is_stable=False)
    out = _regrid(ks, vs)
    return out.reshape(B, GR, GC, 1)
